# Initial kernel scaffold; baseline (speedup 1.0000x reference)
import jax
import jax.numpy as jnp
from jax.experimental import pallas as pl


def kernel(user_id, category_hist, user_table, cat_table, W1, b1, W2, b2, gamma, beta):
    raise NotImplementedError("write your pallas kernel here")


# speedup vs baseline: 3.6242x; 3.6242x over previous
import functools

import jax
import jax.numpy as jnp
from jax import lax
from jax.experimental import pallas as pl
from jax.experimental.pallas import tpu as pltpu
from jax.experimental.pallas import tpu_sc as plsc

B = 4096
HIST = 200
UD = 64
CD = 32
HIDDEN = 256
OUT = 128

NC = 2
NS = 16
NW = NC * NS
BPW = B // NW
C0 = 96
C1 = HIST - C0


def _sc_pool_body(uid_hbm, hist_hbm, utab_hbm, ctab_hbm, u_hbm, c_hbm,
                  uidx_v, urows_v, hidx_v, crows_v, pooled_v, sem):
    wid = lax.axis_index("s") * NC + lax.axis_index("c")
    base = wid * BPW

    pltpu.sync_copy(uid_hbm.at[pl.ds(base, BPW)], uidx_v)
    pltpu.async_copy(utab_hbm.at[uidx_v], urows_v, sem).wait()
    pltpu.sync_copy(urows_v, u_hbm.at[pl.ds(base, BPW)])

    pltpu.sync_copy(hist_hbm.at[pl.ds(base * HIST, BPW * HIST)], hidx_v)

    inv = jnp.float32(1.0 / HIST)

    @pl.loop(0, BPW)
    def _(s):
        d0 = pltpu.async_copy(ctab_hbm.at[hidx_v.at[pl.ds(s * HIST, C0)]],
                              crows_v.at[pl.ds(0, C0)], sem)
        d1 = pltpu.async_copy(ctab_hbm.at[hidx_v.at[pl.ds(s * HIST + C0, C1)]],
                              crows_v.at[pl.ds(C0, C1)], sem)
        d0.wait()
        d1.wait()

        def body(i, acc):
            a0, a1 = acc
            return (a0 + crows_v[i, 0:16], a1 + crows_v[i, 16:32])

        zero = jnp.zeros((16,), jnp.float32)
        a0, a1 = lax.fori_loop(0, HIST, body, (zero, zero))
        pooled_v[s, 0:16] = a0 * inv
        pooled_v[s, 16:32] = a1 * inv

    pltpu.sync_copy(pooled_v, c_hbm.at[pl.ds(base, BPW)])


@jax.jit
def _sc_gather_pool(user_id, hist_flat, user_table, cat_table):
    mesh = plsc.VectorSubcoreMesh(core_axis_name="c", subcore_axis_name="s")
    fn = pl.kernel(
        _sc_pool_body,
        out_type=[jax.ShapeDtypeStruct((B, UD), jnp.float32),
                  jax.ShapeDtypeStruct((B, CD), jnp.float32)],
        mesh=mesh,
        compiler_params=pltpu.CompilerParams(use_tc_tiling_on_sc=False),
        scratch_types=[
            pltpu.VMEM((BPW,), jnp.int32),
            pltpu.VMEM((BPW, UD), jnp.float32),
            pltpu.VMEM((BPW * HIST,), jnp.int32),
            pltpu.VMEM((HIST, CD), jnp.float32),
            pltpu.VMEM((BPW, CD), jnp.float32),
            pltpu.SemaphoreType.DMA,
        ],
    )
    return fn(user_id, hist_flat, user_table, cat_table)


def _mlp_body(u_ref, c_ref, w1u_ref, w1c_ref, b1_ref, w2_ref, b2_ref,
              g_ref, bt_ref, o_ref):
    h = jnp.dot(u_ref[...], w1u_ref[...], preferred_element_type=jnp.float32)
    h = h + jnp.dot(c_ref[...], w1c_ref[...], preferred_element_type=jnp.float32)
    h = jnp.maximum(h + b1_ref[...], 0.0)
    h2 = jnp.dot(h, w2_ref[...], preferred_element_type=jnp.float32) + b2_ref[...]
    mean = jnp.mean(h2, axis=-1, keepdims=True)
    cen = h2 - mean
    var = jnp.mean(cen * cen, axis=-1, keepdims=True)
    o_ref[...] = cen * lax.rsqrt(var + 1e-5) * g_ref[...] + bt_ref[...]


BLK = 512


@jax.jit
def _tc_mlp(u, c, W1u, W1c, b1, W2, b2, gamma, beta):
    grid = (B // BLK,)
    return pl.pallas_call(
        _mlp_body,
        grid=grid,
        in_specs=[
            pl.BlockSpec((BLK, UD), lambda i: (i, 0)),
            pl.BlockSpec((BLK, CD), lambda i: (i, 0)),
            pl.BlockSpec((UD, HIDDEN), lambda i: (0, 0)),
            pl.BlockSpec((CD, HIDDEN), lambda i: (0, 0)),
            pl.BlockSpec((1, HIDDEN), lambda i: (0, 0)),
            pl.BlockSpec((HIDDEN, OUT), lambda i: (0, 0)),
            pl.BlockSpec((1, OUT), lambda i: (0, 0)),
            pl.BlockSpec((1, OUT), lambda i: (0, 0)),
            pl.BlockSpec((1, OUT), lambda i: (0, 0)),
        ],
        out_specs=pl.BlockSpec((BLK, OUT), lambda i: (i, 0)),
        out_shape=jax.ShapeDtypeStruct((B, OUT), jnp.float32),
    )(u, c, W1u, W1c, b1, W2, b2, gamma, beta)


def kernel(user_id, category_hist, user_table, cat_table, W1, b1, W2, b2,
           gamma, beta):
    hist_flat = category_hist.reshape(-1)
    u, c = _sc_gather_pool(user_id, hist_flat, user_table, cat_table)
    return _tc_mlp(u, c, W1[:UD], W1[UD:], b1.reshape(1, -1), W2,
                   b2.reshape(1, -1), gamma.reshape(1, -1),
                   beta.reshape(1, -1))


# speedup vs baseline: 6.4616x; 1.7829x over previous
import functools

import jax
import jax.numpy as jnp
from jax import lax
from jax.experimental import pallas as pl
from jax.experimental.pallas import tpu as pltpu
from jax.experimental.pallas import tpu_sc as plsc

B = 4096
HIST = 200
UD = 64
CD = 32
HIDDEN = 256
OUT = 128

NC = 2
NS = 16
NW = NC * NS
BPW = B // NW
C0 = 96
C1 = HIST - C0


UG = 256
UK = 16


def _sc_user_body(uid_hbm, utab_hbm, u_hbm, uidx_s, sem, dsem):
    cid = lax.axis_index("c")
    half = B // NC
    base = cid * half

    @pl.loop(0, half, step=UG)
    def _(off):
        pltpu.sync_copy(uid_hbm.at[pl.ds(base + off, UG)], uidx_s)

        @pl.loop(0, UG, step=UK)
        def _(g):
            for j in range(UK):
                pltpu.async_copy(utab_hbm.at[uidx_s[g + j]],
                                 u_hbm.at[base + off + g + j], dsem)

        @pl.loop(0, UG, step=UK)
        def _(g):
            for j in range(UK):
                pltpu.make_async_copy(utab_hbm.at[0],
                                      u_hbm.at[base + off + g + j],
                                      dsem).wait()


@jax.jit
def _sc_user_gather(user_id, user_table):
    mesh = plsc.ScalarSubcoreMesh(axis_name="c", num_cores=NC)
    fn = pl.kernel(
        _sc_user_body,
        out_type=jax.ShapeDtypeStruct((B, UD), jnp.float32),
        mesh=mesh,
        scratch_types=[
            pltpu.SMEM((UG,), jnp.int32),
            pltpu.SemaphoreType.DMA,
            pltpu.SemaphoreType.DMA,
        ],
    )
    return fn(user_id, user_table)


def _sc_pool_body(hist_hbm, ctab_hbm, c_hbm, hidx_v, crows_v, pooled_v, sem):
    wid = lax.axis_index("s") * NC + lax.axis_index("c")
    base = wid * BPW

    pltpu.sync_copy(hist_hbm.at[pl.ds(base * HIST, BPW * HIST)], hidx_v)

    inv = jnp.float32(1.0 / HIST)

    @pl.loop(0, BPW)
    def _(s):
        d0 = pltpu.async_copy(ctab_hbm.at[hidx_v.at[pl.ds(s * HIST, C0)]],
                              crows_v.at[pl.ds(0, C0)], sem)
        d1 = pltpu.async_copy(ctab_hbm.at[hidx_v.at[pl.ds(s * HIST + C0, C1)]],
                              crows_v.at[pl.ds(C0, C1)], sem)
        d0.wait()
        d1.wait()

        def body(i, acc):
            a0, a1 = acc
            return (a0 + crows_v[i, 0:16], a1 + crows_v[i, 16:32])

        zero = jnp.zeros((16,), jnp.float32)
        a0, a1 = lax.fori_loop(0, HIST, body, (zero, zero))
        pooled_v[s, 0:16] = a0 * inv
        pooled_v[s, 16:32] = a1 * inv

    pltpu.sync_copy(pooled_v, c_hbm.at[pl.ds(base, BPW)])


@jax.jit
def _sc_cat_pool(hist_flat, cat_table):
    mesh = plsc.VectorSubcoreMesh(core_axis_name="c", subcore_axis_name="s")
    fn = pl.kernel(
        _sc_pool_body,
        out_type=jax.ShapeDtypeStruct((B, CD), jnp.float32),
        mesh=mesh,
        compiler_params=pltpu.CompilerParams(use_tc_tiling_on_sc=False),
        scratch_types=[
            pltpu.VMEM((BPW * HIST,), jnp.int32),
            pltpu.VMEM((HIST, CD), jnp.float32),
            pltpu.VMEM((BPW, CD), jnp.float32),
            pltpu.SemaphoreType.DMA,
        ],
    )
    return fn(hist_flat, cat_table)


def _mlp_body(u_ref, c_ref, w1u_ref, w1c_ref, b1_ref, w2_ref, b2_ref,
              g_ref, bt_ref, o_ref):
    h = jnp.dot(u_ref[...], w1u_ref[...], preferred_element_type=jnp.float32)
    h = h + jnp.dot(c_ref[...], w1c_ref[...], preferred_element_type=jnp.float32)
    h = jnp.maximum(h + b1_ref[...], 0.0)
    h2 = jnp.dot(h, w2_ref[...], preferred_element_type=jnp.float32) + b2_ref[...]
    mean = jnp.mean(h2, axis=-1, keepdims=True)
    cen = h2 - mean
    var = jnp.mean(cen * cen, axis=-1, keepdims=True)
    o_ref[...] = cen * lax.rsqrt(var + 1e-5) * g_ref[...] + bt_ref[...]


BLK = 512


@jax.jit
def _tc_mlp(u, c, W1u, W1c, b1, W2, b2, gamma, beta):
    grid = (B // BLK,)
    return pl.pallas_call(
        _mlp_body,
        grid=grid,
        in_specs=[
            pl.BlockSpec((BLK, UD), lambda i: (i, 0)),
            pl.BlockSpec((BLK, CD), lambda i: (i, 0)),
            pl.BlockSpec((UD, HIDDEN), lambda i: (0, 0)),
            pl.BlockSpec((CD, HIDDEN), lambda i: (0, 0)),
            pl.BlockSpec((1, HIDDEN), lambda i: (0, 0)),
            pl.BlockSpec((HIDDEN, OUT), lambda i: (0, 0)),
            pl.BlockSpec((1, OUT), lambda i: (0, 0)),
            pl.BlockSpec((1, OUT), lambda i: (0, 0)),
            pl.BlockSpec((1, OUT), lambda i: (0, 0)),
        ],
        out_specs=pl.BlockSpec((BLK, OUT), lambda i: (i, 0)),
        out_shape=jax.ShapeDtypeStruct((B, OUT), jnp.float32),
    )(u, c, W1u, W1c, b1, W2, b2, gamma, beta)


def kernel(user_id, category_hist, user_table, cat_table, W1, b1, W2, b2,
           gamma, beta):
    hist_flat = category_hist.reshape(-1)
    u = _sc_user_gather(user_id, user_table)
    c = _sc_cat_pool(hist_flat, cat_table)
    return _tc_mlp(u, c, W1[:UD], W1[UD:], b1.reshape(1, -1), W2,
                   b2.reshape(1, -1), gamma.reshape(1, -1),
                   beta.reshape(1, -1))
